# baseline (speedup 1.0000x reference)
import functools

import jax
import jax.numpy as jnp
from jax import lax
from jax.experimental import pallas as pl
from jax.experimental.pallas import tpu as pltpu
from jax.experimental.pallas import tpu_sc as plsc

H = 32
H2 = H * H
D_EDGE = 16
EB = 1600
QB = EB // 4


def _node_mlp_body(x_ref, w1, b1, w2, b2, w3, b3, o_ref):
    h = jnp.maximum(jnp.dot(x_ref[...], w1[...],
                            preferred_element_type=jnp.float32) + b1[...], 0.0)
    h = jnp.maximum(jnp.dot(h, w2[...],
                            preferred_element_type=jnp.float32) + b2[...], 0.0)
    o_ref[...] = jnp.dot(h, w3[...],
                         preferred_element_type=jnp.float32) + b3[...]


def _node_mlp(x, p):
    n = x.shape[0]
    return pl.pallas_call(
        _node_mlp_body,
        out_shape=jax.ShapeDtypeStruct((n, H), jnp.float32),
    )(x, p['in_W1'], p['in_b1'].reshape(1, H),
      p['in_W2'], p['in_b2'].reshape(1, H),
      p['in_W3'], p['in_b3'].reshape(1, H))


def _edge_msg_body(ea_ref, xj_ref, w1, b1, w2, b2, w3, b3, s_ref, o_ref):
    xj4 = xj_ref[...]
    xj = jnp.concatenate([xj4[:, j * H:(j + 1) * H] for j in range(4)],
                         axis=0)
    ea8 = ea_ref[...]
    ea = jnp.concatenate(
        [ea8[:, j * D_EDGE:(j + 1) * D_EDGE] for j in range(8)], axis=0)
    e = jnp.maximum(jnp.dot(ea, w1[...],
                            preferred_element_type=jnp.float32) + b1[...], 0.0)
    e = jnp.maximum(jnp.dot(e, w2[...],
                            preferred_element_type=jnp.float32) + b2[...], 0.0)
    we = jnp.dot(e.astype(jnp.bfloat16), w3[...],
                 preferred_element_type=jnp.float32).astype(jnp.bfloat16)
    xt = jnp.tile(xj.astype(jnp.bfloat16), (1, H))
    msg = jnp.dot(xt * we, s_ref[...],
                  preferred_element_type=jnp.float32) + jnp.dot(
        xj, b3[...], preferred_element_type=jnp.float32)
    o_ref[...] = jnp.concatenate(
        [msg[j * QB:(j + 1) * QB, :] for j in range(4)], axis=1)


def _edge_messages(edge_attr, xj, p):
    e_total, d_edge = edge_attr.shape
    eb = EB
    grid = e_total // eb
    cols = jnp.arange(H2, dtype=jnp.int32)
    perm = (cols % H) * H + cols // H
    w3p = p['e_W3'][:, perm].astype(jnp.bfloat16)
    b3mat = p['e_b3'].reshape(H, H)
    smat = (cols[:, None] // H == jnp.arange(H, dtype=jnp.int32)[None, :])
    smat = smat.astype(jnp.bfloat16)

    def bcast(shape):
        return pl.BlockSpec(shape, lambda i: (0,) * len(shape))

    msg4 = pl.pallas_call(
        _edge_msg_body,
        grid=(grid,),
        in_specs=[
            pl.BlockSpec((eb // 8, 128), lambda i: (i, 0)),
            pl.BlockSpec((eb // 4, 128), lambda i: (i, 0)),
            bcast((d_edge, H)), bcast((1, H)),
            bcast((H, H)), bcast((1, H)),
            bcast((H, H2)), bcast((H, H)),
            bcast((H2, H)),
        ],
        out_specs=pl.BlockSpec((eb // 4, 128), lambda i: (i, 0)),
        out_shape=jax.ShapeDtypeStruct((e_total // 4, 128), jnp.float32),
    )(edge_attr.reshape(e_total // 8, 128), xj.reshape(e_total // 4, 128),
      p['e_W1'], p['e_b1'].reshape(1, H),
      p['e_W2'], p['e_b2'].reshape(1, H), w3p, b3mat, smat)
    return msg4.reshape(e_total, H)


def _gather_rows(table, idx):
    e_total = idx.shape[0]
    info = plsc.get_sparse_core_info()
    nc, ns = info.num_cores, info.num_subcores
    nw = nc * ns
    per_w = e_total // nw
    chunk = 1000
    n_chunks = per_w // chunk
    mesh = plsc.VectorSubcoreMesh(core_axis_name="c", subcore_axis_name="s")

    @functools.partial(
        pl.kernel, mesh=mesh,
        out_type=jax.ShapeDtypeStruct((e_total, H), jnp.float32),
        compiler_params=pltpu.CompilerParams(use_tc_tiling_on_sc=False),
        scratch_types=[
            pltpu.VMEM((chunk,), jnp.int32),
            pltpu.VMEM((chunk, H), jnp.float32),
            pltpu.SemaphoreType.DMA,
        ],
    )
    def k(table_hbm, idx_hbm, out_hbm, idx_v, rows_v, sem):
        wid = lax.axis_index("s") * nc + lax.axis_index("c")
        base = wid * per_w
        for j in range(n_chunks):
            off = base + j * chunk
            pltpu.sync_copy(idx_hbm.at[pl.ds(off, chunk)], idx_v)
            pltpu.async_copy(table_hbm.at[idx_v], rows_v, sem).wait()
            pltpu.sync_copy(rows_v, out_hbm.at[pl.ds(off, chunk)])

    return k(table, idx)


def _scatter_add(msg, dst, n_pad):
    e_total = msg.shape[0]
    info = plsc.get_sparse_core_info()
    nc, ns = info.num_cores, info.num_subcores
    nw = nc * ns
    per_w = e_total // nw
    chunk = 1000
    n_chunks = per_w // chunk
    rows_per_tile = n_pad // ns
    zeros = jnp.zeros((n_pad, H), jnp.float32)
    mesh = plsc.VectorSubcoreMesh(core_axis_name="c", subcore_axis_name="s")

    @functools.partial(
        pl.kernel, mesh=mesh,
        out_type=jax.ShapeDtypeStruct((nc, n_pad, H), jnp.float32),
        compiler_params=pltpu.CompilerParams(use_tc_tiling_on_sc=False),
        scratch_types=[
            pltpu.VMEM((chunk,), jnp.int32),
            pltpu.VMEM((chunk, H), jnp.float32),
            pltpu.VMEM((rows_per_tile, H), jnp.float32),
            pltpu.VMEM_SHARED((n_pad, H), jnp.float32),
        ],
    )
    def k(msg_hbm, dst_hbm, zero_hbm, out_hbm, idx_v, msg_v, bounce_v,
          accum_sh):
        cid = lax.axis_index("c")
        sid = lax.axis_index("s")
        wid = sid * nc + cid
        r0 = sid * rows_per_tile
        pltpu.sync_copy(zero_hbm.at[pl.ds(r0, rows_per_tile)], bounce_v)
        pltpu.sync_copy(bounce_v, accum_sh.at[pl.ds(r0, rows_per_tile)])
        plsc.subcore_barrier()
        base = wid * per_w
        for j in range(n_chunks):
            off = base + j * chunk
            pltpu.sync_copy(dst_hbm.at[pl.ds(off, chunk)], idx_v)
            pltpu.sync_copy(msg_hbm.at[pl.ds(off, chunk)], msg_v)
            pltpu.sync_copy(msg_v, accum_sh.at[idx_v], add=True)
        plsc.subcore_barrier()
        pltpu.sync_copy(accum_sh.at[pl.ds(r0, rows_per_tile)], bounce_v)
        pltpu.sync_copy(bounce_v, out_hbm.at[cid, pl.ds(r0, rows_per_tile)])

    return k(msg, dst, zeros)


def _gru_body(a0, a1, h_ref, root, cb, wr, br, wz, bz, wn, bn, bhn, o_ref):
    m = a0[...] + a1[...] + jnp.dot(
        h_ref[...], root[...], preferred_element_type=jnp.float32) + cb[...]
    r = jax.nn.sigmoid(jnp.dot(m, wr[...],
                               preferred_element_type=jnp.float32) + br[...])
    z = jax.nn.sigmoid(jnp.dot(m, wz[...],
                               preferred_element_type=jnp.float32) + bz[...])
    n = jnp.tanh(jnp.dot(m, wn[...], preferred_element_type=jnp.float32)
                 + bn[...] + r * bhn[...])
    o_ref[...] = (1.0 - z) * n


def _gru(agg0, agg1, h, p):
    n_nodes = h.shape[0]
    wi = p['gru_Wi']
    bi = p['gru_bi']
    bh = p['gru_bh']
    return pl.pallas_call(
        _gru_body,
        out_shape=jax.ShapeDtypeStruct((n_nodes, H), jnp.float32),
    )(agg0, agg1, h, p['root'], p['conv_bias'].reshape(1, H),
      wi[:, 0:H], (bi[0:H] + bh[0:H]).reshape(1, H),
      wi[:, H:2 * H], (bi[H:2 * H] + bh[H:2 * H]).reshape(1, H),
      wi[:, 2 * H:], bi[2 * H:].reshape(1, H), bh[2 * H:].reshape(1, H))


def kernel(x, edge_index, edge_attr, params):
    n_nodes = x.shape[0]
    n_pad = ((n_nodes + 1279) // 1280) * 1280
    e_total = edge_attr.shape[0]
    def _perm(ix):
        return ix.reshape(e_total // EB, EB // 8, 4, 2).transpose(
            0, 3, 1, 2).reshape(-1)

    src = _perm(edge_index[0].astype(jnp.int32))
    dst = _perm(edge_index[1].astype(jnp.int32))

    h = _node_mlp(x, params)
    xj = _gather_rows(h, src)
    msg = _edge_messages(edge_attr, xj, params)
    parts = _scatter_add(msg, dst, n_pad)
    return _gru(parts[0, :n_nodes], parts[1, :n_nodes], h, params)


# speedup vs baseline: 1.9028x; 1.9028x over previous
import functools

import jax
import jax.numpy as jnp
from jax import lax
from jax.experimental import pallas as pl
from jax.experimental.pallas import tpu as pltpu
from jax.experimental.pallas import tpu_sc as plsc

H = 32
H2 = H * H
D_EDGE = 16
EB = 3200
QB = EB // 4


def _node_mlp_body(x_ref, w1, b1, w2, b2, w3, b3, o_ref):
    h = jnp.maximum(jnp.dot(x_ref[...], w1[...],
                            preferred_element_type=jnp.float32) + b1[...], 0.0)
    h = jnp.maximum(jnp.dot(h, w2[...],
                            preferred_element_type=jnp.float32) + b2[...], 0.0)
    o_ref[...] = jnp.dot(h, w3[...],
                         preferred_element_type=jnp.float32) + b3[...]


def _node_mlp(x, p):
    n = x.shape[0]
    return pl.pallas_call(
        _node_mlp_body,
        out_shape=jax.ShapeDtypeStruct((n, H), jnp.float32),
    )(x, p['in_W1'], p['in_b1'].reshape(1, H),
      p['in_W2'], p['in_b2'].reshape(1, H),
      p['in_W3'], p['in_b3'].reshape(1, H))


def _edge_msg_body(ea_ref, xj_ref, w1, b1, w2, b2, w3, b3, s_ref, o_ref):
    xj4 = xj_ref[...]
    xj = jnp.concatenate([xj4[:, j * H:(j + 1) * H] for j in range(4)],
                         axis=0)
    e = jnp.maximum(jnp.dot(ea_ref[...], w1[...],
                            preferred_element_type=jnp.float32) + b1[...], 0.0)
    e = jnp.maximum(jnp.dot(e, w2[...],
                            preferred_element_type=jnp.float32) + b2[...], 0.0)
    we = jnp.dot(e.astype(jnp.bfloat16), w3[...],
                 preferred_element_type=jnp.float32).astype(jnp.bfloat16)
    xt = jnp.tile(xj.astype(jnp.bfloat16), (1, H))
    msg = jnp.dot(xt * we, s_ref[...],
                  preferred_element_type=jnp.float32) + jnp.dot(
        xj, b3[...], preferred_element_type=jnp.float32)
    o_ref[...] = jnp.concatenate(
        [msg[j * QB:(j + 1) * QB, :] for j in range(4)], axis=1)


def _edge_messages(edge_attr, xj, p):
    e_total, d_edge = edge_attr.shape
    eb = EB
    grid = e_total // eb
    cols = jnp.arange(H2, dtype=jnp.int32)
    perm = (cols % H) * H + cols // H
    w3p = p['e_W3'][:, perm].astype(jnp.bfloat16)
    b3mat = p['e_b3'].reshape(H, H)
    smat = (cols[:, None] // H == jnp.arange(H, dtype=jnp.int32)[None, :])
    smat = smat.astype(jnp.bfloat16)

    def bcast(shape):
        return pl.BlockSpec(shape, lambda i: (0,) * len(shape))

    msg4 = pl.pallas_call(
        _edge_msg_body,
        grid=(grid,),
        in_specs=[
            pl.BlockSpec((eb, d_edge), lambda i: (i, 0)),
            pl.BlockSpec((eb // 4, 128), lambda i: (i, 0)),
            bcast((d_edge, H)), bcast((1, H)),
            bcast((H, H)), bcast((1, H)),
            bcast((H, H2)), bcast((H, H)),
            bcast((H2, H)),
        ],
        out_specs=pl.BlockSpec((eb // 4, 128), lambda i: (i, 0)),
        out_shape=jax.ShapeDtypeStruct((e_total // 4, 128), jnp.float32),
    )(edge_attr, xj.reshape(e_total // 4, 128),
      p['e_W1'], p['e_b1'].reshape(1, H),
      p['e_W2'], p['e_b2'].reshape(1, H), w3p, b3mat, smat)
    return msg4.reshape(e_total, H)


def _gather_rows(table, idx):
    e_total = idx.shape[0]
    info = plsc.get_sparse_core_info()
    nc, ns = info.num_cores, info.num_subcores
    nw = nc * ns
    per_w = e_total // nw
    chunk = 1000
    n_chunks = per_w // chunk
    mesh = plsc.VectorSubcoreMesh(core_axis_name="c", subcore_axis_name="s")

    @functools.partial(
        pl.kernel, mesh=mesh,
        out_type=jax.ShapeDtypeStruct((e_total, H), jnp.float32),
        compiler_params=pltpu.CompilerParams(use_tc_tiling_on_sc=False),
        scratch_types=[
            pltpu.VMEM((chunk,), jnp.int32),
            pltpu.VMEM((chunk, H), jnp.float32),
            pltpu.SemaphoreType.DMA,
        ],
    )
    def k(table_hbm, idx_hbm, out_hbm, idx_v, rows_v, sem):
        wid = lax.axis_index("s") * nc + lax.axis_index("c")
        base = wid * per_w
        for j in range(n_chunks):
            off = base + j * chunk
            pltpu.sync_copy(idx_hbm.at[pl.ds(off, chunk)], idx_v)
            pltpu.async_copy(table_hbm.at[idx_v], rows_v, sem).wait()
            pltpu.sync_copy(rows_v, out_hbm.at[pl.ds(off, chunk)])

    return k(table, idx)


def _scatter_add(msg, dst, n_pad):
    e_total = msg.shape[0]
    info = plsc.get_sparse_core_info()
    nc, ns = info.num_cores, info.num_subcores
    nw = nc * ns
    per_w = e_total // nw
    chunk = 1000
    n_chunks = per_w // chunk
    rows_per_tile = n_pad // ns
    zeros = jnp.zeros((n_pad, H), jnp.float32)
    mesh = plsc.VectorSubcoreMesh(core_axis_name="c", subcore_axis_name="s")

    @functools.partial(
        pl.kernel, mesh=mesh,
        out_type=jax.ShapeDtypeStruct((nc, n_pad, H), jnp.float32),
        compiler_params=pltpu.CompilerParams(use_tc_tiling_on_sc=False),
        scratch_types=[
            pltpu.VMEM((chunk,), jnp.int32),
            pltpu.VMEM((chunk, H), jnp.float32),
            pltpu.VMEM((rows_per_tile, H), jnp.float32),
            pltpu.VMEM_SHARED((n_pad, H), jnp.float32),
        ],
    )
    def k(msg_hbm, dst_hbm, zero_hbm, out_hbm, idx_v, msg_v, bounce_v,
          accum_sh):
        cid = lax.axis_index("c")
        sid = lax.axis_index("s")
        wid = sid * nc + cid
        r0 = sid * rows_per_tile
        pltpu.sync_copy(zero_hbm.at[pl.ds(r0, rows_per_tile)], bounce_v)
        pltpu.sync_copy(bounce_v, accum_sh.at[pl.ds(r0, rows_per_tile)])
        plsc.subcore_barrier()
        base = wid * per_w
        for j in range(n_chunks):
            off = base + j * chunk
            pltpu.sync_copy(dst_hbm.at[pl.ds(off, chunk)], idx_v)
            pltpu.sync_copy(msg_hbm.at[pl.ds(off, chunk)], msg_v)
            pltpu.sync_copy(msg_v, accum_sh.at[idx_v], add=True)
        plsc.subcore_barrier()
        pltpu.sync_copy(accum_sh.at[pl.ds(r0, rows_per_tile)], bounce_v)
        pltpu.sync_copy(bounce_v, out_hbm.at[cid, pl.ds(r0, rows_per_tile)])

    return k(msg, dst, zeros)


def _gru_body(a0, a1, h_ref, root, cb, wr, br, wz, bz, wn, bn, bhn, o_ref):
    m = a0[...] + a1[...] + jnp.dot(
        h_ref[...], root[...], preferred_element_type=jnp.float32) + cb[...]
    r = jax.nn.sigmoid(jnp.dot(m, wr[...],
                               preferred_element_type=jnp.float32) + br[...])
    z = jax.nn.sigmoid(jnp.dot(m, wz[...],
                               preferred_element_type=jnp.float32) + bz[...])
    n = jnp.tanh(jnp.dot(m, wn[...], preferred_element_type=jnp.float32)
                 + bn[...] + r * bhn[...])
    o_ref[...] = (1.0 - z) * n


def _gru(agg0, agg1, h, p):
    n_nodes = h.shape[0]
    wi = p['gru_Wi']
    bi = p['gru_bi']
    bh = p['gru_bh']
    return pl.pallas_call(
        _gru_body,
        out_shape=jax.ShapeDtypeStruct((n_nodes, H), jnp.float32),
    )(agg0, agg1, h, p['root'], p['conv_bias'].reshape(1, H),
      wi[:, 0:H], (bi[0:H] + bh[0:H]).reshape(1, H),
      wi[:, H:2 * H], (bi[H:2 * H] + bh[H:2 * H]).reshape(1, H),
      wi[:, 2 * H:], bi[2 * H:].reshape(1, H), bh[2 * H:].reshape(1, H))


def kernel(x, edge_index, edge_attr, params):
    n_nodes = x.shape[0]
    n_pad = ((n_nodes + 1279) // 1280) * 1280
    e_total = edge_attr.shape[0]
    def _perm(ix):
        return ix.reshape(e_total // EB, 4, QB).transpose(0, 2, 1).reshape(-1)

    src = _perm(edge_index[0].astype(jnp.int32))
    dst = _perm(edge_index[1].astype(jnp.int32))

    h = _node_mlp(x, params)
    xj = _gather_rows(h, src)
    msg = _edge_messages(edge_attr, xj, params)
    parts = _scatter_add(msg, dst, n_pad)
    return _gru(parts[0, :n_nodes], parts[1, :n_nodes], h, params)


# speedup vs baseline: 1.9382x; 1.0186x over previous
import functools

import jax
import jax.numpy as jnp
from jax import lax
from jax.experimental import pallas as pl
from jax.experimental.pallas import tpu as pltpu
from jax.experimental.pallas import tpu_sc as plsc

H = 32
H2 = H * H
D_EDGE = 16
EB = 6400
QB = EB // 4


def _node_mlp_body(x_ref, w1, b1, w2, b2, w3, b3, o_ref):
    h = jnp.maximum(jnp.dot(x_ref[...], w1[...],
                            preferred_element_type=jnp.float32) + b1[...], 0.0)
    h = jnp.maximum(jnp.dot(h, w2[...],
                            preferred_element_type=jnp.float32) + b2[...], 0.0)
    o_ref[...] = jnp.dot(h, w3[...],
                         preferred_element_type=jnp.float32) + b3[...]


def _node_mlp(x, p):
    n = x.shape[0]
    return pl.pallas_call(
        _node_mlp_body,
        out_shape=jax.ShapeDtypeStruct((n, H), jnp.float32),
    )(x, p['in_W1'], p['in_b1'].reshape(1, H),
      p['in_W2'], p['in_b2'].reshape(1, H),
      p['in_W3'], p['in_b3'].reshape(1, H))


def _edge_msg_body(ea_ref, xj_ref, w1, b1, w2, b2, w3, b3, s_ref, o_ref):
    xj4 = xj_ref[...]
    xj = jnp.concatenate([xj4[:, j * H:(j + 1) * H] for j in range(4)],
                         axis=0)
    e = jnp.maximum(jnp.dot(ea_ref[...], w1[...],
                            preferred_element_type=jnp.float32) + b1[...], 0.0)
    e = jnp.maximum(jnp.dot(e, w2[...],
                            preferred_element_type=jnp.float32) + b2[...], 0.0)
    we = jnp.dot(e.astype(jnp.bfloat16), w3[...],
                 preferred_element_type=jnp.float32).astype(jnp.bfloat16)
    xt = jnp.tile(xj.astype(jnp.bfloat16), (1, H))
    msg = jnp.dot(xt * we, s_ref[...],
                  preferred_element_type=jnp.float32) + jnp.dot(
        xj, b3[...], preferred_element_type=jnp.float32)
    o_ref[...] = jnp.concatenate(
        [msg[j * QB:(j + 1) * QB, :] for j in range(4)], axis=1)


def _edge_messages(edge_attr, xj, p):
    e_total, d_edge = edge_attr.shape
    eb = EB
    grid = e_total // eb
    cols = jnp.arange(H2, dtype=jnp.int32)
    perm = (cols % H) * H + cols // H
    w3p = p['e_W3'][:, perm].astype(jnp.bfloat16)
    b3mat = p['e_b3'].reshape(H, H)
    smat = (cols[:, None] // H == jnp.arange(H, dtype=jnp.int32)[None, :])
    smat = smat.astype(jnp.bfloat16)

    def bcast(shape):
        return pl.BlockSpec(shape, lambda i: (0,) * len(shape))

    msg4 = pl.pallas_call(
        _edge_msg_body,
        grid=(grid,),
        in_specs=[
            pl.BlockSpec((eb, d_edge), lambda i: (i, 0)),
            pl.BlockSpec((eb // 4, 128), lambda i: (i, 0)),
            bcast((d_edge, H)), bcast((1, H)),
            bcast((H, H)), bcast((1, H)),
            bcast((H, H2)), bcast((H, H)),
            bcast((H2, H)),
        ],
        out_specs=pl.BlockSpec((eb // 4, 128), lambda i: (i, 0)),
        out_shape=jax.ShapeDtypeStruct((e_total // 4, 128), jnp.float32),
    )(edge_attr, xj.reshape(e_total // 4, 128),
      p['e_W1'], p['e_b1'].reshape(1, H),
      p['e_W2'], p['e_b2'].reshape(1, H), w3p, b3mat, smat)
    return msg4.reshape(e_total, H)


def _gather_rows(table, idx):
    e_total = idx.shape[0]
    info = plsc.get_sparse_core_info()
    nc, ns = info.num_cores, info.num_subcores
    nw = nc * ns
    per_w = e_total // nw
    chunk = 1000
    n_chunks = per_w // chunk
    mesh = plsc.VectorSubcoreMesh(core_axis_name="c", subcore_axis_name="s")

    @functools.partial(
        pl.kernel, mesh=mesh,
        out_type=jax.ShapeDtypeStruct((e_total, H), jnp.float32),
        compiler_params=pltpu.CompilerParams(use_tc_tiling_on_sc=False),
        scratch_types=[
            pltpu.VMEM((chunk,), jnp.int32),
            pltpu.VMEM((chunk, H), jnp.float32),
            pltpu.SemaphoreType.DMA,
        ],
    )
    def k(table_hbm, idx_hbm, out_hbm, idx_v, rows_v, sem):
        wid = lax.axis_index("s") * nc + lax.axis_index("c")
        base = wid * per_w
        for j in range(n_chunks):
            off = base + j * chunk
            pltpu.sync_copy(idx_hbm.at[pl.ds(off, chunk)], idx_v)
            pltpu.async_copy(table_hbm.at[idx_v], rows_v, sem).wait()
            pltpu.sync_copy(rows_v, out_hbm.at[pl.ds(off, chunk)])

    return k(table, idx)


def _scatter_add(msg, dst, n_pad):
    e_total = msg.shape[0]
    info = plsc.get_sparse_core_info()
    nc, ns = info.num_cores, info.num_subcores
    nw = nc * ns
    per_w = e_total // nw
    chunk = 1000
    n_chunks = per_w // chunk
    rows_per_tile = n_pad // ns
    zeros = jnp.zeros((n_pad, H), jnp.float32)
    mesh = plsc.VectorSubcoreMesh(core_axis_name="c", subcore_axis_name="s")

    @functools.partial(
        pl.kernel, mesh=mesh,
        out_type=jax.ShapeDtypeStruct((nc, n_pad, H), jnp.float32),
        compiler_params=pltpu.CompilerParams(use_tc_tiling_on_sc=False),
        scratch_types=[
            pltpu.VMEM((chunk,), jnp.int32),
            pltpu.VMEM((chunk, H), jnp.float32),
            pltpu.VMEM((rows_per_tile, H), jnp.float32),
            pltpu.VMEM_SHARED((n_pad, H), jnp.float32),
        ],
    )
    def k(msg_hbm, dst_hbm, zero_hbm, out_hbm, idx_v, msg_v, bounce_v,
          accum_sh):
        cid = lax.axis_index("c")
        sid = lax.axis_index("s")
        wid = sid * nc + cid
        r0 = sid * rows_per_tile
        pltpu.sync_copy(zero_hbm.at[pl.ds(r0, rows_per_tile)], bounce_v)
        pltpu.sync_copy(bounce_v, accum_sh.at[pl.ds(r0, rows_per_tile)])
        plsc.subcore_barrier()
        base = wid * per_w
        for j in range(n_chunks):
            off = base + j * chunk
            pltpu.sync_copy(dst_hbm.at[pl.ds(off, chunk)], idx_v)
            pltpu.sync_copy(msg_hbm.at[pl.ds(off, chunk)], msg_v)
            pltpu.sync_copy(msg_v, accum_sh.at[idx_v], add=True)
        plsc.subcore_barrier()
        pltpu.sync_copy(accum_sh.at[pl.ds(r0, rows_per_tile)], bounce_v)
        pltpu.sync_copy(bounce_v, out_hbm.at[cid, pl.ds(r0, rows_per_tile)])

    return k(msg, dst, zeros)


def _gru_body(a0, a1, h_ref, root, cb, wr, br, wz, bz, wn, bn, bhn, o_ref):
    m = a0[...] + a1[...] + jnp.dot(
        h_ref[...], root[...], preferred_element_type=jnp.float32) + cb[...]
    r = jax.nn.sigmoid(jnp.dot(m, wr[...],
                               preferred_element_type=jnp.float32) + br[...])
    z = jax.nn.sigmoid(jnp.dot(m, wz[...],
                               preferred_element_type=jnp.float32) + bz[...])
    n = jnp.tanh(jnp.dot(m, wn[...], preferred_element_type=jnp.float32)
                 + bn[...] + r * bhn[...])
    o_ref[...] = (1.0 - z) * n


def _gru(agg0, agg1, h, p):
    n_nodes = h.shape[0]
    wi = p['gru_Wi']
    bi = p['gru_bi']
    bh = p['gru_bh']
    return pl.pallas_call(
        _gru_body,
        out_shape=jax.ShapeDtypeStruct((n_nodes, H), jnp.float32),
    )(agg0, agg1, h, p['root'], p['conv_bias'].reshape(1, H),
      wi[:, 0:H], (bi[0:H] + bh[0:H]).reshape(1, H),
      wi[:, H:2 * H], (bi[H:2 * H] + bh[H:2 * H]).reshape(1, H),
      wi[:, 2 * H:], bi[2 * H:].reshape(1, H), bh[2 * H:].reshape(1, H))


def kernel(x, edge_index, edge_attr, params):
    n_nodes = x.shape[0]
    n_pad = ((n_nodes + 1279) // 1280) * 1280
    e_total = edge_attr.shape[0]
    def _perm(ix):
        return ix.reshape(e_total // EB, 4, QB).transpose(0, 2, 1).reshape(-1)

    src = _perm(edge_index[0].astype(jnp.int32))
    dst = _perm(edge_index[1].astype(jnp.int32))

    h = _node_mlp(x, params)
    xj = _gather_rows(h, src)
    msg = _edge_messages(edge_attr, xj, params)
    parts = _scatter_add(msg, dst, n_pad)
    return _gru(parts[0, :n_nodes], parts[1, :n_nodes], h, params)
